# baseline (speedup 1.0000x reference)
import functools

import jax
import jax.numpy as jnp
from jax import lax
from jax.experimental import pallas as pl
from jax.experimental.pallas import tpu as pltpu
from jax.experimental.pallas import tpu_sc as plsc

B, S, F = 1024, 200, 64
SD, WD = 32, 32
GD = SD + WD
OUT_D = F + GD
N = B * S
NUM_CORES = 2
NUM_SUBCORES = 16
NW = NUM_CORES * NUM_SUBCORES
TOK_W = N // NW
C = 128
GROUP = 640
SUB = GROUP // C
ITERS = TOK_W // GROUP

_mesh = plsc.VectorSubcoreMesh(core_axis_name="c", subcore_axis_name="s")


@functools.partial(
    pl.kernel,
    mesh=_mesh,
    compiler_params=pltpu.CompilerParams(use_tc_tiling_on_sc=False),
    out_type=jax.ShapeDtypeStruct((N, OUT_D), jnp.float32),
    scratch_types=[
        pltpu.VMEM((TOK_W,), jnp.int32),
        pltpu.VMEM((TOK_W,), jnp.int32),
        pltpu.VMEM((2, GROUP, SD), jnp.float32),
        pltpu.VMEM((2, GROUP, WD), jnp.float32),
        pltpu.SemaphoreType.DMA((2,)),
        pltpu.SemaphoreType.DMA((2,)),
        pltpu.SemaphoreType.DMA,
    ],
)
def _emb_kernel(feat_hbm, sids_hbm, wids_hbm, stab_hbm, wtab_hbm, out_hbm,
                widx_v, sidx_v, srows_v, wrows_v, in_sem, out_sem, feat_sem):
    wid = lax.axis_index("s") * NUM_CORES + lax.axis_index("c")
    base0 = wid * TOK_W

    pltpu.async_copy(
        feat_hbm.at[pl.ds(base0, TOK_W)],
        out_hbm.at[pl.ds(base0, TOK_W), pl.ds(0, F)], feat_sem)

    pltpu.sync_copy(wids_hbm.at[pl.ds(base0, TOK_W)], widx_v)
    pltpu.sync_copy(sids_hbm.at[pl.ds(base0, TOK_W)], sidx_v)

    def start_in(g, p):
        for j in range(SUB):
            o = g * GROUP + j * C
            pltpu.async_copy(
                stab_hbm.at[sidx_v.at[pl.ds(o, C)]],
                srows_v.at[p, pl.ds(j * C, C)], in_sem.at[p])
            pltpu.async_copy(
                wtab_hbm.at[widx_v.at[pl.ds(o, C)]],
                wrows_v.at[p, pl.ds(j * C, C)], in_sem.at[p])

    def wait_in(p):
        pltpu.make_async_copy(
            stab_hbm.at[pl.ds(0, GROUP)], srows_v.at[p], in_sem.at[p]).wait()
        pltpu.make_async_copy(
            wtab_hbm.at[pl.ds(0, GROUP)], wrows_v.at[p], in_sem.at[p]).wait()

    def start_out(g, p):
        base = base0 + g * GROUP
        pltpu.async_copy(
            srows_v.at[p], out_hbm.at[pl.ds(base, GROUP), pl.ds(F, SD)],
            out_sem.at[p])
        pltpu.async_copy(
            wrows_v.at[p], out_hbm.at[pl.ds(base, GROUP), pl.ds(F + SD, WD)],
            out_sem.at[p])

    def wait_out(g, p):
        base = base0 + g * GROUP
        pltpu.make_async_copy(
            srows_v.at[p], out_hbm.at[pl.ds(base, GROUP), pl.ds(F, SD)],
            out_sem.at[p]).wait()
        pltpu.make_async_copy(
            wrows_v.at[p], out_hbm.at[pl.ds(base, GROUP), pl.ds(F + SD, WD)],
            out_sem.at[p]).wait()

    start_in(0, 0)

    def body(g, carry):
        p = lax.rem(g, 2)
        q = 1 - p

        @pl.when(g >= 1)
        def _():
            wait_out(g - 1, q)

        @pl.when(g < ITERS - 1)
        def _():
            start_in(g + 1, q)

        wait_in(p)
        start_out(g, p)
        return carry

    lax.fori_loop(0, ITERS, body, 0)
    wait_out(ITERS - 1, lax.rem(ITERS - 1, 2))
    pltpu.make_async_copy(
        feat_hbm.at[pl.ds(base0, TOK_W)],
        out_hbm.at[pl.ds(base0, TOK_W), pl.ds(0, F)], feat_sem).wait()


def kernel(feature_tensor, shape_ids, word_ids, shape_table, word_table):
    feat = feature_tensor.reshape(N, F)
    sids = shape_ids.reshape(N).astype(jnp.int32)
    wids = word_ids.reshape(N).astype(jnp.int32)
    out = _emb_kernel(feat, sids, wids, shape_table, word_table)
    return out.reshape(B, S, OUT_D)


# speedup vs baseline: 2.1724x; 2.1724x over previous
import functools

import jax
import jax.numpy as jnp
from jax import lax
from jax.experimental import pallas as pl
from jax.experimental.pallas import tpu as pltpu
from jax.experimental.pallas import tpu_sc as plsc

B, S, F = 1024, 200, 64
SD, WD = 32, 32
OUT_D = F + SD + WD
N = B * S
SHAPE_V, WORD_V = 1001, 1000001
WORD_VP = 1000004
WBLK = 1024
WGRID = (WORD_V + WBLK - 1) // WBLK
NUM_CORES = 2
NUM_SUBCORES = 16
NW = NUM_CORES * NUM_SUBCORES
TOK_W = N // NW
C = 128
ITERS = TOK_W // C

_mesh = plsc.VectorSubcoreMesh(core_axis_name="c", subcore_axis_name="s")


def _repack_wt_body(wt_ref, eye_ref, out_ref):
    x = wt_ref[...]
    y = lax.dot_general(x, eye_ref[...], (((0,), (0,)), ((), ())),
                        preferred_element_type=jnp.float32)
    y3 = y.reshape(WBLK // 4, 4, SD)
    for g in range(4):
        out_ref[:, g * SD:(g + 1) * SD] = y3[:, g, :]


def _repack_word_table(word_table):
    return pl.pallas_call(
        _repack_wt_body,
        grid=(WGRID,),
        in_specs=[
            pl.BlockSpec((SD, WBLK), lambda i: (0, i)),
            pl.BlockSpec((SD, SD), lambda i: (0, 0)),
        ],
        out_specs=pl.BlockSpec((WBLK // 4, 128), lambda i: (i, 0)),
        out_shape=jax.ShapeDtypeStruct((WGRID * WBLK // 4, 128), jnp.float32),
    )(word_table.T, jnp.eye(SD, dtype=jnp.float32))


@functools.partial(
    pl.kernel,
    mesh=_mesh,
    compiler_params=pltpu.CompilerParams(use_tc_tiling_on_sc=False),
    out_type=jax.ShapeDtypeStruct((N, OUT_D), jnp.float32),
    scratch_types=[
        pltpu.VMEM((TOK_W,), jnp.int32),
        pltpu.VMEM((TOK_W,), jnp.int32),
        pltpu.VMEM((2, C, F), jnp.float32),
        pltpu.VMEM((2, C, SD), jnp.float32),
        pltpu.VMEM((2, C, WD), jnp.float32),
        pltpu.SemaphoreType.DMA((2,)),
        pltpu.SemaphoreType.DMA((2,)),
    ],
)
def _emb_kernel(feat_hbm, sids_hbm, wids_hbm, stab_hbm, wtab_hbm, out_hbm,
                widx_v, sidx_v, feat_v, srows_v, wrows_v, in_sem, out_sem):
    wid = lax.axis_index("s") * NUM_CORES + lax.axis_index("c")
    base0 = wid * TOK_W
    pltpu.sync_copy(wids_hbm.at[pl.ds(base0, TOK_W)], widx_v)
    pltpu.sync_copy(sids_hbm.at[pl.ds(base0, TOK_W)], sidx_v)

    def start_in(c, p):
        base = base0 + c * C
        pltpu.async_copy(feat_hbm.at[pl.ds(base, C)], feat_v.at[p], in_sem.at[p])
        pltpu.async_copy(
            stab_hbm.at[sidx_v.at[pl.ds(c * C, C)]], srows_v.at[p], in_sem.at[p])
        pltpu.async_copy(
            wtab_hbm.at[widx_v.at[pl.ds(c * C, C)]], wrows_v.at[p], in_sem.at[p])

    def wait_in(p):
        pltpu.make_async_copy(
            feat_hbm.at[pl.ds(0, C)], feat_v.at[p], in_sem.at[p]).wait()
        pltpu.make_async_copy(
            stab_hbm.at[pl.ds(0, C)], srows_v.at[p], in_sem.at[p]).wait()
        pltpu.make_async_copy(
            wtab_hbm.at[pl.ds(0, C)], wrows_v.at[p], in_sem.at[p]).wait()

    def start_out(c, p):
        base = base0 + c * C
        pltpu.async_copy(
            feat_v.at[p], out_hbm.at[pl.ds(base, C), pl.ds(0, F)], out_sem.at[p])
        pltpu.async_copy(
            srows_v.at[p], out_hbm.at[pl.ds(base, C), pl.ds(F, SD)], out_sem.at[p])
        pltpu.async_copy(
            wrows_v.at[p], out_hbm.at[pl.ds(base, C), pl.ds(F + SD, WD)],
            out_sem.at[p])

    def wait_out(c, p):
        base = base0 + c * C
        pltpu.make_async_copy(
            feat_v.at[p], out_hbm.at[pl.ds(base, C), pl.ds(0, F)],
            out_sem.at[p]).wait()
        pltpu.make_async_copy(
            srows_v.at[p], out_hbm.at[pl.ds(base, C), pl.ds(F, SD)],
            out_sem.at[p]).wait()
        pltpu.make_async_copy(
            wrows_v.at[p], out_hbm.at[pl.ds(base, C), pl.ds(F + SD, WD)],
            out_sem.at[p]).wait()

    start_in(0, 0)

    def body(it, carry):
        p = lax.rem(it, 2)
        q = 1 - p

        @pl.when(it >= 1)
        def _():
            wait_out(it - 1, q)

        @pl.when(it < ITERS - 1)
        def _():
            start_in(it + 1, q)

        wait_in(p)
        start_out(it, p)
        return carry

    lax.fori_loop(0, ITERS, body, 0)
    wait_out(ITERS - 1, lax.rem(ITERS - 1, 2))


def kernel(feature_tensor, shape_ids, word_ids, shape_table, word_table):
    feat = feature_tensor.reshape(N, F)
    wtab = _repack_word_table(word_table).reshape(WGRID * WBLK, WD)
    sids = shape_ids.reshape(N).astype(jnp.int32)
    wids = word_ids.reshape(N).astype(jnp.int32)
    out = _emb_kernel(feat, sids, wids, shape_table, wtab)
    return out.reshape(B, S, OUT_D)


# speedup vs baseline: 3.2436x; 1.4931x over previous
import functools

import jax
import jax.numpy as jnp
from jax import lax
from jax.experimental import pallas as pl
from jax.experimental.pallas import tpu as pltpu
from jax.experimental.pallas import tpu_sc as plsc

B, S, F = 1024, 200, 64
SD, WD = 32, 32
OUT_D = F + SD + WD
N = B * S
NUM_CORES = 2
NUM_SUBCORES = 16
NW = NUM_CORES * NUM_SUBCORES
TOK_W = N // NW
C = 128
ITERS = TOK_W // C

_mesh = plsc.VectorSubcoreMesh(core_axis_name="c", subcore_axis_name="s")


@functools.partial(
    pl.kernel,
    mesh=_mesh,
    compiler_params=pltpu.CompilerParams(use_tc_tiling_on_sc=False),
    out_type=jax.ShapeDtypeStruct((N, OUT_D), jnp.float32),
    scratch_types=[
        pltpu.VMEM((TOK_W,), jnp.int32),
        pltpu.VMEM((TOK_W,), jnp.int32),
        pltpu.VMEM((2, C, F), jnp.float32),
        pltpu.VMEM((2, C, SD), jnp.float32),
        pltpu.VMEM((2, C, WD), jnp.float32),
        pltpu.SemaphoreType.DMA((2,)),
        pltpu.SemaphoreType.DMA((2,)),
    ],
)
def _emb_kernel(feat_hbm, sids_hbm, wids_hbm, stab_hbm, wtab_hbm, out_hbm,
                widx_v, sidx_v, feat_v, srows_v, wrows_v, in_sem, out_sem):
    wid = lax.axis_index("s") * NUM_CORES + lax.axis_index("c")
    base0 = wid * TOK_W
    pltpu.sync_copy(wids_hbm.at[pl.ds(base0, TOK_W)], widx_v)
    pltpu.sync_copy(sids_hbm.at[pl.ds(base0, TOK_W)], sidx_v)

    def start_in(c, p):
        base = base0 + c * C
        pltpu.async_copy(feat_hbm.at[pl.ds(base, C)], feat_v.at[p], in_sem.at[p])
        pltpu.async_copy(
            stab_hbm.at[sidx_v.at[pl.ds(c * C, C)]], srows_v.at[p], in_sem.at[p])
        pltpu.async_copy(
            wtab_hbm.at[widx_v.at[pl.ds(c * C, C)]], wrows_v.at[p], in_sem.at[p])

    def wait_in(p):
        pltpu.make_async_copy(
            feat_hbm.at[pl.ds(0, C)], feat_v.at[p], in_sem.at[p]).wait()
        pltpu.make_async_copy(
            stab_hbm.at[pl.ds(0, C)], srows_v.at[p], in_sem.at[p]).wait()
        pltpu.make_async_copy(
            wtab_hbm.at[pl.ds(0, C)], wrows_v.at[p], in_sem.at[p]).wait()

    def start_out(c, p):
        base = base0 + c * C
        pltpu.async_copy(
            feat_v.at[p], out_hbm.at[pl.ds(base, C), pl.ds(0, F)], out_sem.at[p])
        pltpu.async_copy(
            srows_v.at[p], out_hbm.at[pl.ds(base, C), pl.ds(F, SD)], out_sem.at[p])
        pltpu.async_copy(
            wrows_v.at[p], out_hbm.at[pl.ds(base, C), pl.ds(F + SD, WD)],
            out_sem.at[p])

    def wait_out(c, p):
        base = base0 + c * C
        pltpu.make_async_copy(
            feat_v.at[p], out_hbm.at[pl.ds(base, C), pl.ds(0, F)],
            out_sem.at[p]).wait()
        pltpu.make_async_copy(
            srows_v.at[p], out_hbm.at[pl.ds(base, C), pl.ds(F, SD)],
            out_sem.at[p]).wait()
        pltpu.make_async_copy(
            wrows_v.at[p], out_hbm.at[pl.ds(base, C), pl.ds(F + SD, WD)],
            out_sem.at[p]).wait()

    start_in(0, 0)

    def body(it, carry):
        p = lax.rem(it, 2)
        q = 1 - p

        @pl.when(it >= 1)
        def _():
            wait_out(it - 1, q)

        @pl.when(it < ITERS - 1)
        def _():
            start_in(it + 1, q)

        wait_in(p)
        start_out(it, p)
        return carry

    lax.fori_loop(0, ITERS, body, 0)
    wait_out(ITERS - 1, lax.rem(ITERS - 1, 2))


def kernel(feature_tensor, shape_ids, word_ids, shape_table, word_table):
    feat = feature_tensor.reshape(N, F)
    sids = shape_ids.reshape(N).astype(jnp.int32)
    wids = word_ids.reshape(N).astype(jnp.int32)
    out = _emb_kernel(feat, sids, wids, shape_table, word_table)
    return out.reshape(B, S, OUT_D)
